# baseline (speedup 1.0000x reference)
import functools

import jax
import jax.numpy as jnp
from jax import lax
from jax.experimental import pallas as pl
from jax.experimental.pallas import tpu as pltpu
from jax.experimental.pallas import tpu_sc as plsc

B, K, D = 16, 2048, 2048
TOPK = 64
KC_BLK = 256
NUM_KC = K // KC_BLK
D2 = D // 2


def _scores_body(emb_ref, w_ref, s_ref):
    xbf = emb_ref[0].astype(jnp.bfloat16)
    wbf = w_ref[0].astype(jnp.bfloat16)
    res = lax.dot_general(xbf, wbf.reshape(D, 1), (((1,), (0,)), ((), ())),
                          preferred_element_type=jnp.float32)
    s_ref[...] = res.reshape(1, 1, KC_BLK)


def _scores(embeddings, w2):
    out = pl.pallas_call(
        _scores_body,
        grid=(B * NUM_KC,),
        in_specs=[
            pl.BlockSpec((1, KC_BLK, D),
                         lambda i: (i // NUM_KC, i % NUM_KC, 0)),
            pl.BlockSpec((1, D), lambda i: (0, 0)),
        ],
        out_specs=pl.BlockSpec((1, 1, KC_BLK), lambda i: (i, 0, 0)),
        out_shape=jax.ShapeDtypeStruct((B * NUM_KC, 1, KC_BLK), jnp.float32),
        compiler_params=pltpu.CompilerParams(
            dimension_semantics=("arbitrary",),
        ),
    )(embeddings, w2)
    return out.reshape(B, K)


def _topk_body(s_ref, maskf_ref, weights_ref, gid_ref):
    neg_inf = jnp.float32(-jnp.inf)
    s = jnp.where(maskf_ref[...] > 0, s_ref[...], neg_inf)
    iota_k = lax.broadcasted_iota(jnp.int32, (B, K), 1)
    iota_g = lax.broadcasted_iota(jnp.int32, (B, 2 * TOPK), 1)
    brow = lax.broadcasted_iota(jnp.int32, (B, 1), 0)
    w_acc = jnp.zeros((B, K), jnp.float32)
    gids = jnp.zeros((B, 2 * TOPK), jnp.int32)
    inv_k = jnp.float32(1.0 / TOPK)
    for j in range(TOPK):
        m = jnp.max(s, axis=1, keepdims=True)
        idx = jnp.min(jnp.where(s == m, iota_k, K), axis=1,
                      keepdims=True)
        sel = iota_k == idx
        s = jnp.where(sel, neg_inf, s)
        w_acc = w_acc + jnp.where(sel, inv_k, jnp.float32(0.0))
        gid2 = (brow * K + idx) * 2
        gids = jnp.where(iota_g == j, gid2, gids)
        gids = jnp.where(iota_g == TOPK + j, gid2 + 1, gids)
    weights_ref[...] = w_acc
    gid_ref[...] = gids


def _topk(scores, maskf):
    return pl.pallas_call(
        _topk_body,
        out_shape=[
            jax.ShapeDtypeStruct((B, K), jnp.float32),
            jax.ShapeDtypeStruct((B, 2 * TOPK), jnp.int32),
        ],
    )(scores, maskf)


def _gather_mean_body(table_hbm, gid_hbm, out_hbm, idx_v, rows_v, acc_v, sem):
    nc = plsc.get_sparse_core_info().num_cores
    wid = lax.axis_index("s") * nc + lax.axis_index("c")
    b = wid // 2
    h = wid % 2
    base = b * (2 * TOPK) + h * TOPK
    pltpu.sync_copy(gid_hbm.at[pl.ds(base, TOPK)], idx_v)
    pltpu.async_copy(table_hbm.at[idx_v], rows_v, sem).wait()
    inv_k = jnp.float32(1.0 / TOPK)
    grp = 8
    for g in range(D2 // (16 * grp)):
        def body(r, accs):
            return tuple(
                accs[i] + rows_v[r, pl.ds((g * grp + i) * 16, 16)]
                for i in range(grp))
        accs = lax.fori_loop(
            0, TOPK, body,
            tuple(jnp.zeros((16,), jnp.float32) for _ in range(grp)))
        for i in range(grp):
            acc_v[pl.ds((g * grp + i) * 16, 16)] = accs[i] * inv_k
    pltpu.sync_copy(acc_v, out_hbm.at[b, pl.ds(h * D2, D2)])


@functools.cache
def _gather_mean():
    return pl.kernel(
        _gather_mean_body,
        mesh=plsc.VectorSubcoreMesh(core_axis_name="c", subcore_axis_name="s"),
        out_type=jax.ShapeDtypeStruct((B, D), jnp.float32),
        scratch_types=[
            pltpu.VMEM((TOPK,), jnp.int32),
            pltpu.VMEM((TOPK, D2), jnp.float32),
            pltpu.VMEM((D2,), jnp.float32),
            pltpu.SemaphoreType.DMA,
        ],
    )


def kernel(embeddings, mask, W, b):
    maskf = mask.astype(jnp.float32)
    w2 = W.reshape(1, D)
    scores = _scores(embeddings, w2)
    weights, gid = _topk(scores, maskf)
    table = embeddings.reshape(B * K * 2, D2)
    pooled = _gather_mean()(table, gid.reshape(B * 2 * TOPK))
    return pooled, weights


# speedup vs baseline: 3.0869x; 3.0869x over previous
import functools

import jax
import jax.numpy as jnp
from jax import lax
from jax.experimental import pallas as pl
from jax.experimental.pallas import tpu as pltpu
from jax.experimental.pallas import tpu_sc as plsc

B, K, D = 16, 2048, 2048
TOPK = 64
KC_BLK = 512
NUM_KC = K // KC_BLK
D2 = D // 2


def _score_topk_body(emb_ref, w_ref, maskf_ref, weights_ref, gid_ref,
                     scores_ref):
    b = pl.program_id(0)
    kc = pl.program_id(1)
    xbf = emb_ref[0].astype(jnp.bfloat16)
    wbf = w_ref[0].astype(jnp.bfloat16)
    res = lax.dot_general(xbf, wbf.reshape(D, 1), (((1,), (0,)), ((), ())),
                          preferred_element_type=jnp.float32)
    scores_ref[pl.ds(b, 1), pl.ds(kc * KC_BLK, KC_BLK)] = res.reshape(1, KC_BLK)

    @pl.when(jnp.logical_and(b == B - 1, kc == NUM_KC - 1))
    def _():
        neg_inf = jnp.float32(-jnp.inf)
        s = jnp.where(maskf_ref[...] > 0, scores_ref[...], neg_inf)
        iota_k = lax.broadcasted_iota(jnp.int32, (B, K), 1)
        iota_g = lax.broadcasted_iota(jnp.int32, (B, 2 * TOPK), 1)
        w_acc = jnp.zeros((B, K), jnp.float32)
        gids = jnp.zeros((B, 2 * TOPK), jnp.int32)
        inv_k = jnp.float32(1.0 / TOPK)
        for j in range(TOPK):
            m = jnp.max(s, axis=1, keepdims=True)
            idx = jnp.min(jnp.where(s == m, iota_k, K), axis=1,
                          keepdims=True)
            sel = iota_k == idx
            s = jnp.where(sel, neg_inf, s)
            w_acc = w_acc + jnp.where(sel, inv_k, jnp.float32(0.0))
            gids = jnp.where(iota_g == j, idx, gids)
            gids = jnp.where(iota_g == TOPK + j, idx, gids)
        weights_ref[...] = w_acc
        gid_ref[...] = gids


def _score_topk(embeddings, w2, maskf):
    return pl.pallas_call(
        _score_topk_body,
        grid=(B, NUM_KC),
        in_specs=[
            pl.BlockSpec((1, KC_BLK, D), lambda b, kc: (b, kc, 0)),
            pl.BlockSpec((1, D), lambda b, kc: (0, 0)),
            pl.BlockSpec((B, K), lambda b, kc: (0, 0)),
        ],
        out_specs=[
            pl.BlockSpec((B, K), lambda b, kc: (0, 0)),
            pl.BlockSpec((B, 2 * TOPK), lambda b, kc: (0, 0)),
        ],
        out_shape=[
            jax.ShapeDtypeStruct((B, K), jnp.float32),
            jax.ShapeDtypeStruct((B, 2 * TOPK), jnp.int32),
        ],
        scratch_shapes=[pltpu.VMEM((B, K), jnp.float32)],
        compiler_params=pltpu.CompilerParams(
            dimension_semantics=("arbitrary", "arbitrary"),
        ),
    )(embeddings, w2, maskf)


def _gather_mean_body(emb_hbm, gid_hbm, out_hbm, idx_v, rows_v, acc_v, sem):
    nc = plsc.get_sparse_core_info().num_cores
    wid = lax.axis_index("s") * nc + lax.axis_index("c")
    b = wid // 2
    h = wid % 2
    base = b * (2 * TOPK) + h * TOPK
    pltpu.sync_copy(gid_hbm.at[pl.ds(base, TOPK)], idx_v)
    pltpu.async_copy(emb_hbm.at[b].at[idx_v, pl.ds(h * D2, D2)],
                     rows_v, sem).wait()
    inv_k = jnp.float32(1.0 / TOPK)
    grp = 8
    for g in range(D2 // (16 * grp)):
        def body(r, accs):
            return tuple(
                accs[i] + rows_v[r, pl.ds((g * grp + i) * 16, 16)]
                for i in range(grp))
        accs = lax.fori_loop(
            0, TOPK, body,
            tuple(jnp.zeros((16,), jnp.float32) for _ in range(grp)))
        for i in range(grp):
            acc_v[pl.ds((g * grp + i) * 16, 16)] = accs[i] * inv_k
    pltpu.sync_copy(acc_v, out_hbm.at[b, pl.ds(h * D2, D2)])


@functools.cache
def _gather_mean():
    return pl.kernel(
        _gather_mean_body,
        mesh=plsc.VectorSubcoreMesh(core_axis_name="c", subcore_axis_name="s"),
        out_type=jax.ShapeDtypeStruct((B, D), jnp.float32),
        scratch_types=[
            pltpu.VMEM((TOPK,), jnp.int32),
            pltpu.VMEM((TOPK, D2), jnp.float32),
            pltpu.VMEM((D2,), jnp.float32),
            pltpu.SemaphoreType.DMA,
        ],
    )


def kernel(embeddings, mask, W, b):
    maskf = mask.astype(jnp.float32)
    w2 = W.reshape(1, D)
    weights, gid = _score_topk(embeddings, w2, maskf)
    pooled = _gather_mean()(embeddings, gid.reshape(B * 2 * TOPK))
    return pooled, weights


# speedup vs baseline: 3.6483x; 1.1819x over previous
import functools

import jax
import jax.numpy as jnp
from jax import lax
from jax.experimental import pallas as pl
from jax.experimental.pallas import tpu as pltpu
from jax.experimental.pallas import tpu_sc as plsc

B, K, D = 16, 2048, 2048
TOPK = 64
KC_BLK = 1024
NUM_KC = K // KC_BLK
D2 = D // 2


def _score_topk_body(emb_ref, w_ref, maskf_ref, weights_ref, gid_ref,
                     scores_ref):
    b = pl.program_id(0)
    kc = pl.program_id(1)
    xbf = emb_ref[0].astype(jnp.bfloat16)
    wbf = w_ref[0].astype(jnp.bfloat16)
    res = lax.dot_general(xbf, wbf.reshape(D, 1), (((1,), (0,)), ((), ())),
                          preferred_element_type=jnp.float32)
    scores_ref[pl.ds(b, 1), pl.ds(kc * KC_BLK, KC_BLK)] = res.reshape(1, KC_BLK)

    @pl.when(jnp.logical_and(b == B - 1, kc == NUM_KC - 1))
    def _():
        neg_inf = jnp.float32(-jnp.inf)
        s = jnp.where(maskf_ref[...] > 0, scores_ref[...], neg_inf)
        iota_k = lax.broadcasted_iota(jnp.int32, (B, K), 1)
        iota_g = lax.broadcasted_iota(jnp.int32, (B, 2 * TOPK), 1)
        w_acc = jnp.zeros((B, K), jnp.float32)
        gids = jnp.zeros((B, 2 * TOPK), jnp.int32)
        inv_k = jnp.float32(1.0 / TOPK)
        for j in range(TOPK):
            m = jnp.max(s, axis=1, keepdims=True)
            idx = jnp.min(jnp.where(s == m, iota_k, K), axis=1,
                          keepdims=True)
            sel = iota_k == idx
            s = jnp.where(sel, neg_inf, s)
            w_acc = w_acc + jnp.where(sel, inv_k, jnp.float32(0.0))
            gids = jnp.where(iota_g == j, idx, gids)
            gids = jnp.where(iota_g == TOPK + j, idx, gids)
        weights_ref[...] = w_acc
        gid_ref[...] = gids


def _score_topk(embeddings, w2, maskf):
    return pl.pallas_call(
        _score_topk_body,
        grid=(B, NUM_KC),
        in_specs=[
            pl.BlockSpec((1, KC_BLK, D), lambda b, kc: (b, kc, 0)),
            pl.BlockSpec((1, D), lambda b, kc: (0, 0)),
            pl.BlockSpec((B, K), lambda b, kc: (0, 0)),
        ],
        out_specs=[
            pl.BlockSpec((B, K), lambda b, kc: (0, 0)),
            pl.BlockSpec((B, 2 * TOPK), lambda b, kc: (0, 0)),
        ],
        out_shape=[
            jax.ShapeDtypeStruct((B, K), jnp.float32),
            jax.ShapeDtypeStruct((B, 2 * TOPK), jnp.int32),
        ],
        scratch_shapes=[pltpu.VMEM((B, K), jnp.float32)],
        compiler_params=pltpu.CompilerParams(
            dimension_semantics=("arbitrary", "arbitrary"),
        ),
    )(embeddings, w2, maskf)


def _gather_mean_body(emb_hbm, gid_hbm, out_hbm, idx_v, rows_v, acc_v, sem):
    nc = plsc.get_sparse_core_info().num_cores
    wid = lax.axis_index("s") * nc + lax.axis_index("c")
    b = wid // 2
    h = wid % 2
    base = b * (2 * TOPK) + h * TOPK
    pltpu.sync_copy(gid_hbm.at[pl.ds(base, TOPK)], idx_v)
    pltpu.async_copy(emb_hbm.at[b].at[idx_v, pl.ds(h * D2, D2)],
                     rows_v, sem).wait()
    inv_k = jnp.float32(1.0 / TOPK)
    grp = 8
    for g in range(D2 // (16 * grp)):
        def body(r, accs):
            return tuple(
                accs[i] + rows_v[r, pl.ds((g * grp + i) * 16, 16)]
                for i in range(grp))
        accs = lax.fori_loop(
            0, TOPK, body,
            tuple(jnp.zeros((16,), jnp.float32) for _ in range(grp)))
        for i in range(grp):
            acc_v[pl.ds((g * grp + i) * 16, 16)] = accs[i] * inv_k
    pltpu.sync_copy(acc_v, out_hbm.at[b, pl.ds(h * D2, D2)])


@functools.cache
def _gather_mean():
    return pl.kernel(
        _gather_mean_body,
        mesh=plsc.VectorSubcoreMesh(core_axis_name="c", subcore_axis_name="s"),
        out_type=jax.ShapeDtypeStruct((B, D), jnp.float32),
        scratch_types=[
            pltpu.VMEM((TOPK,), jnp.int32),
            pltpu.VMEM((TOPK, D2), jnp.float32),
            pltpu.VMEM((D2,), jnp.float32),
            pltpu.SemaphoreType.DMA,
        ],
    )


def kernel(embeddings, mask, W, b):
    maskf = mask.astype(jnp.float32)
    w2 = W.reshape(1, D)
    weights, gid = _score_topk(embeddings, w2, maskf)
    pooled = _gather_mean()(embeddings, gid.reshape(B * 2 * TOPK))
    return pooled, weights


# speedup vs baseline: 3.6510x; 1.0007x over previous
import functools

import jax
import jax.numpy as jnp
from jax import lax
from jax.experimental import pallas as pl
from jax.experimental.pallas import tpu as pltpu
from jax.experimental.pallas import tpu_sc as plsc

B, K, D = 16, 2048, 2048
TOPK = 64
KC_BLK = 2048
NUM_KC = K // KC_BLK
D2 = D // 2


def _score_topk_body(emb_ref, w_ref, maskf_ref, weights_ref, gid_ref,
                     scores_ref):
    b = pl.program_id(0)
    kc = pl.program_id(1)
    xbf = emb_ref[0].astype(jnp.bfloat16)
    wbf = w_ref[0].astype(jnp.bfloat16)
    res = lax.dot_general(xbf, wbf.reshape(D, 1), (((1,), (0,)), ((), ())),
                          preferred_element_type=jnp.float32)
    scores_ref[pl.ds(b, 1), pl.ds(kc * KC_BLK, KC_BLK)] = res.reshape(1, KC_BLK)

    @pl.when(jnp.logical_and(b == B - 1, kc == NUM_KC - 1))
    def _():
        neg_inf = jnp.float32(-jnp.inf)
        s = jnp.where(maskf_ref[...] > 0, scores_ref[...], neg_inf)
        iota_k = lax.broadcasted_iota(jnp.int32, (B, K), 1)
        iota_g = lax.broadcasted_iota(jnp.int32, (B, 2 * TOPK), 1)
        w_acc = jnp.zeros((B, K), jnp.float32)
        gids = jnp.zeros((B, 2 * TOPK), jnp.int32)
        inv_k = jnp.float32(1.0 / TOPK)
        for j in range(TOPK):
            m = jnp.max(s, axis=1, keepdims=True)
            idx = jnp.min(jnp.where(s == m, iota_k, K), axis=1,
                          keepdims=True)
            sel = iota_k == idx
            s = jnp.where(sel, neg_inf, s)
            w_acc = w_acc + jnp.where(sel, inv_k, jnp.float32(0.0))
            gids = jnp.where(iota_g == j, idx, gids)
            gids = jnp.where(iota_g == TOPK + j, idx, gids)
        weights_ref[...] = w_acc
        gid_ref[...] = gids


def _score_topk(embeddings, w2, maskf):
    return pl.pallas_call(
        _score_topk_body,
        grid=(B, NUM_KC),
        in_specs=[
            pl.BlockSpec((1, KC_BLK, D), lambda b, kc: (b, kc, 0)),
            pl.BlockSpec((1, D), lambda b, kc: (0, 0)),
            pl.BlockSpec((B, K), lambda b, kc: (0, 0)),
        ],
        out_specs=[
            pl.BlockSpec((B, K), lambda b, kc: (0, 0)),
            pl.BlockSpec((B, 2 * TOPK), lambda b, kc: (0, 0)),
        ],
        out_shape=[
            jax.ShapeDtypeStruct((B, K), jnp.float32),
            jax.ShapeDtypeStruct((B, 2 * TOPK), jnp.int32),
        ],
        scratch_shapes=[pltpu.VMEM((B, K), jnp.float32)],
        compiler_params=pltpu.CompilerParams(
            dimension_semantics=("arbitrary", "arbitrary"),
        ),
    )(embeddings, w2, maskf)


def _gather_mean_body(emb_hbm, gid_hbm, out_hbm, idx_v, rows_v, acc_v, sem):
    nc = plsc.get_sparse_core_info().num_cores
    wid = lax.axis_index("s") * nc + lax.axis_index("c")
    b = wid // 2
    h = wid % 2
    base = b * (2 * TOPK) + h * TOPK
    pltpu.sync_copy(gid_hbm.at[pl.ds(base, TOPK)], idx_v)
    pltpu.async_copy(emb_hbm.at[b].at[idx_v, pl.ds(h * D2, D2)],
                     rows_v, sem).wait()
    inv_k = jnp.float32(1.0 / TOPK)
    grp = 8
    for g in range(D2 // (16 * grp)):
        def body(r, accs):
            return tuple(
                accs[i] + rows_v[r, pl.ds((g * grp + i) * 16, 16)]
                for i in range(grp))
        accs = lax.fori_loop(
            0, TOPK, body,
            tuple(jnp.zeros((16,), jnp.float32) for _ in range(grp)))
        for i in range(grp):
            acc_v[pl.ds((g * grp + i) * 16, 16)] = accs[i] * inv_k
    pltpu.sync_copy(acc_v, out_hbm.at[b, pl.ds(h * D2, D2)])


@functools.cache
def _gather_mean():
    return pl.kernel(
        _gather_mean_body,
        mesh=plsc.VectorSubcoreMesh(core_axis_name="c", subcore_axis_name="s"),
        out_type=jax.ShapeDtypeStruct((B, D), jnp.float32),
        scratch_types=[
            pltpu.VMEM((TOPK,), jnp.int32),
            pltpu.VMEM((TOPK, D2), jnp.float32),
            pltpu.VMEM((D2,), jnp.float32),
            pltpu.SemaphoreType.DMA,
        ],
    )


def kernel(embeddings, mask, W, b):
    maskf = mask.astype(jnp.float32)
    w2 = W.reshape(1, D)
    weights, gid = _score_topk(embeddings, w2, maskf)
    pooled = _gather_mean()(embeddings, gid.reshape(B * 2 * TOPK))
    return pooled, weights


# speedup vs baseline: 3.6533x; 1.0006x over previous
import functools

import jax
import jax.numpy as jnp
from jax import lax
from jax.experimental import pallas as pl
from jax.experimental.pallas import tpu as pltpu
from jax.experimental.pallas import tpu_sc as plsc

B, K, D = 16, 2048, 2048
TOPK = 64
KC_BLK = 2048
NUM_KC = K // KC_BLK
D2 = D // 2


def _score_topk_body(emb_ref, w_ref, maskf_ref, weights_ref, gid_ref,
                     scores_ref):
    b = pl.program_id(0)
    kc = pl.program_id(1)
    res = lax.dot_general(emb_ref[0], w_ref[0].reshape(D, 1),
                          (((1,), (0,)), ((), ())),
                          precision=lax.Precision.DEFAULT,
                          preferred_element_type=jnp.float32)
    scores_ref[pl.ds(b, 1), pl.ds(kc * KC_BLK, KC_BLK)] = res.reshape(1, KC_BLK)

    @pl.when(jnp.logical_and(b == B - 1, kc == NUM_KC - 1))
    def _():
        neg_inf = jnp.float32(-jnp.inf)
        s = jnp.where(maskf_ref[...] > 0, scores_ref[...], neg_inf)
        iota_k = lax.broadcasted_iota(jnp.int32, (B, K), 1)
        iota_g = lax.broadcasted_iota(jnp.int32, (B, 2 * TOPK), 1)
        w_acc = jnp.zeros((B, K), jnp.float32)
        gids = jnp.zeros((B, 2 * TOPK), jnp.int32)
        inv_k = jnp.float32(1.0 / TOPK)
        for j in range(TOPK):
            m = jnp.max(s, axis=1, keepdims=True)
            idx = jnp.min(jnp.where(s == m, iota_k, K), axis=1,
                          keepdims=True)
            sel = iota_k == idx
            s = jnp.where(sel, neg_inf, s)
            w_acc = w_acc + jnp.where(sel, inv_k, jnp.float32(0.0))
            gids = jnp.where(iota_g == j, idx, gids)
            gids = jnp.where(iota_g == TOPK + j, idx, gids)
        weights_ref[...] = w_acc
        gid_ref[...] = gids


def _score_topk(embeddings, w2, maskf):
    return pl.pallas_call(
        _score_topk_body,
        grid=(B, NUM_KC),
        in_specs=[
            pl.BlockSpec((1, KC_BLK, D), lambda b, kc: (b, kc, 0)),
            pl.BlockSpec((1, D), lambda b, kc: (0, 0)),
            pl.BlockSpec((B, K), lambda b, kc: (0, 0)),
        ],
        out_specs=[
            pl.BlockSpec((B, K), lambda b, kc: (0, 0)),
            pl.BlockSpec((B, 2 * TOPK), lambda b, kc: (0, 0)),
        ],
        out_shape=[
            jax.ShapeDtypeStruct((B, K), jnp.float32),
            jax.ShapeDtypeStruct((B, 2 * TOPK), jnp.int32),
        ],
        scratch_shapes=[pltpu.VMEM((B, K), jnp.float32)],
        compiler_params=pltpu.CompilerParams(
            dimension_semantics=("arbitrary", "arbitrary"),
        ),
    )(embeddings, w2, maskf)


def _gather_mean_body(emb_hbm, gid_hbm, out_hbm, idx_v, rows_v, acc_v, sem):
    nc = plsc.get_sparse_core_info().num_cores
    wid = lax.axis_index("s") * nc + lax.axis_index("c")
    b = wid // 2
    h = wid % 2
    base = b * (2 * TOPK) + h * TOPK
    pltpu.sync_copy(gid_hbm.at[pl.ds(base, TOPK)], idx_v)
    pltpu.async_copy(emb_hbm.at[b].at[idx_v, pl.ds(h * D2, D2)],
                     rows_v, sem).wait()
    inv_k = jnp.float32(1.0 / TOPK)
    grp = 8
    for g in range(D2 // (16 * grp)):
        def body(r, accs):
            return tuple(
                accs[i] + rows_v[r, pl.ds((g * grp + i) * 16, 16)]
                for i in range(grp))
        accs = lax.fori_loop(
            0, TOPK, body,
            tuple(jnp.zeros((16,), jnp.float32) for _ in range(grp)))
        for i in range(grp):
            acc_v[pl.ds((g * grp + i) * 16, 16)] = accs[i] * inv_k
    pltpu.sync_copy(acc_v, out_hbm.at[b, pl.ds(h * D2, D2)])


@functools.cache
def _gather_mean():
    return pl.kernel(
        _gather_mean_body,
        mesh=plsc.VectorSubcoreMesh(core_axis_name="c", subcore_axis_name="s"),
        out_type=jax.ShapeDtypeStruct((B, D), jnp.float32),
        scratch_types=[
            pltpu.VMEM((TOPK,), jnp.int32),
            pltpu.VMEM((TOPK, D2), jnp.float32),
            pltpu.VMEM((D2,), jnp.float32),
            pltpu.SemaphoreType.DMA,
        ],
    )


def kernel(embeddings, mask, W, b):
    maskf = mask.astype(jnp.float32)
    w2 = W.reshape(1, D)
    weights, gid = _score_topk(embeddings, w2, maskf)
    pooled = _gather_mean()(embeddings, gid.reshape(B * 2 * TOPK))
    return pooled, weights
